# baseline (speedup 1.0000x reference)
import functools

import jax
import jax.numpy as jnp
from jax import lax
from jax.experimental import pallas as pl
from jax.experimental.pallas import tpu as pltpu
from jax.experimental.pallas import tpu_sc as plsc

_LANE = 128
_CHUNK_TILES = 160
_NBUF = 3
_LEAD = 2


def kernel(edge_index):
    two, n_cols = edge_index.shape
    info = plsc.get_sparse_core_info()
    nc, ns = info.num_cores, info.num_subcores
    nw = nc * ns
    total_tiles = -(-n_cols // _LANE)
    tiles_pw = -(-total_tiles // nw)
    n_chunks = -(-tiles_pw // _CHUNK_TILES)
    nbuf = min(_NBUF, n_chunks)
    lead = min(_LEAD, nbuf - 1) if nbuf > 1 else 0
    mesh = plsc.VectorSubcoreMesh(core_axis_name="c", subcore_axis_name="s")

    @functools.partial(
        pl.kernel,
        mesh=mesh,
        out_type=jax.ShapeDtypeStruct((two, n_cols), edge_index.dtype),
        scratch_types=(
            [pltpu.VMEM((two, _CHUNK_TILES * _LANE), jnp.int32)
             for _ in range(nbuf)]
            + [pltpu.SemaphoreType.DMA((nbuf,)),
               pltpu.SemaphoreType.DMA((nbuf,))]
        ),
    )
    def _copy(x_hbm, o_hbm, *rest):
        bufs = rest[:nbuf]
        sin, sout = rest[nbuf], rest[nbuf + 1]
        wid = lax.axis_index("s") * nc + lax.axis_index("c")
        start_tile = jnp.minimum(wid * tiles_pw, total_tiles - tiles_pw)
        base = pl.multiple_of(start_tile * _LANE, _LANE)

        def chunk_cols(j):
            return min(_CHUNK_TILES, tiles_pw - j * _CHUNK_TILES) * _LANE

        def in_dma(j):
            sz = chunk_cols(j)
            return pltpu.make_async_copy(
                x_hbm.at[:, pl.ds(base + j * _CHUNK_TILES * _LANE, sz)],
                bufs[j % nbuf].at[:, pl.ds(0, sz)],
                sin.at[j % nbuf],
            )

        def out_dma(j):
            sz = chunk_cols(j)
            return pltpu.make_async_copy(
                bufs[j % nbuf].at[:, pl.ds(0, sz)],
                o_hbm.at[:, pl.ds(base + j * _CHUNK_TILES * _LANE, sz)],
                sout.at[j % nbuf],
            )

        waited_out = set()
        for j in range(min(lead, n_chunks)):
            in_dma(j).start()
        for j in range(n_chunks):
            in_dma(j).wait()
            nxt = j + lead
            if nxt < n_chunks:
                prev = nxt - nbuf
                if prev >= 0:
                    out_dma(prev).wait()
                    waited_out.add(prev)
                in_dma(nxt).start()
            out_dma(j).start()
        for j in range(n_chunks):
            if j not in waited_out:
                out_dma(j).wait()

    return _copy(edge_index)


# speedup vs baseline: 1.7044x; 1.7044x over previous
import jax
import jax.numpy as jnp
from jax.experimental import pallas as pl


def _copy_body(x_ref, o_ref):
    o_ref[...] = x_ref[...]


def _pick_chunk(n_cols):
    for chunk in (1280000, 640000, 128000, 64000, 32000, 12800, 6400, 1280,
                  128):
        if n_cols % chunk == 0:
            return chunk
    return None


def kernel(edge_index):
    two, n_cols = edge_index.shape
    chunk = _pick_chunk(n_cols)
    if chunk is None:
        chunk = n_cols
    grid = n_cols // chunk
    return pl.pallas_call(
        _copy_body,
        grid=(grid,),
        in_specs=[pl.BlockSpec((two, chunk), lambda i: (0, i))],
        out_specs=pl.BlockSpec((two, chunk), lambda i: (0, i)),
        out_shape=jax.ShapeDtypeStruct(edge_index.shape, edge_index.dtype),
    )(edge_index)
